# baseline (speedup 1.0000x reference)
import jax
import jax.numpy as jnp
from jax.experimental import pallas as pl
from jax.experimental.pallas import tpu as pltpu

_BLK_N = 4096


def _body(nn_ref, q_ref, m_ref, o_ref):
    b = pl.program_id(0)
    j = pl.program_id(1)
    nn = nn_ref[b]
    jmax = (nn - 1) // _BLK_N
    neg = jnp.finfo(jnp.float32).min

    @pl.when(j <= jmax)
    def _valid():
        q = q_ref[0]
        m = m_ref[0]
        logits = jax.lax.dot_general(
            q, m, (((1,), (1,)), ((), ())),
            preferred_element_type=jnp.float32,
        )
        n_idx = j * _BLK_N + jax.lax.broadcasted_iota(
            jnp.int32, logits.shape, 1
        )
        o_ref[0] = jnp.where(n_idx < nn, logits, neg)

    @pl.when(j > jmax)
    def _fill():
        o_ref[0] = jnp.full_like(o_ref[0], neg)


def kernel(query_vector, node_nums, m_A):
    B, Q, D = query_vector.shape
    N = m_A.shape[1]
    nb = pl.cdiv(N, _BLK_N)

    def q_map(b, j, nn_ref):
        return (b, 0, 0)

    def m_map(b, j, nn_ref):
        return (b, j, 0)

    def o_map(b, j, nn_ref):
        return (b, 0, j)

    grid_spec = pltpu.PrefetchScalarGridSpec(
        num_scalar_prefetch=1,
        grid=(B, nb),
        in_specs=[
            pl.BlockSpec((1, Q, D), q_map),
            pl.BlockSpec((1, _BLK_N, D), m_map),
        ],
        out_specs=pl.BlockSpec((1, Q, _BLK_N), o_map),
    )
    return pl.pallas_call(
        _body,
        grid_spec=grid_spec,
        out_shape=jax.ShapeDtypeStruct((B, Q, N), jnp.float32),
        compiler_params=pltpu.CompilerParams(
            dimension_semantics=("parallel", "arbitrary"),
        ),
    )(node_nums.astype(jnp.int32), query_vector, m_A)


# speedup vs baseline: 1.3773x; 1.3773x over previous
import jax
import jax.numpy as jnp
from jax.experimental import pallas as pl
from jax.experimental.pallas import tpu as pltpu

_BLK_N = 10000


def _body(nn_ref, q_ref, m_ref, o_ref):
    b = pl.program_id(0)
    j = pl.program_id(1)
    nn = nn_ref[b]
    jmax = (nn - 1) // _BLK_N
    neg = jnp.finfo(jnp.float32).min

    @pl.when(j <= jmax)
    def _valid():
        q = q_ref[0]
        m = m_ref[0]
        logits = jax.lax.dot_general(
            q, m, (((1,), (1,)), ((), ())),
            preferred_element_type=jnp.float32,
        )
        n_idx = j * _BLK_N + jax.lax.broadcasted_iota(
            jnp.int32, logits.shape, 1
        )
        o_ref[0] = jnp.where(n_idx < nn, logits, neg)

    @pl.when(j > jmax)
    def _fill():
        o_ref[0] = jnp.full_like(o_ref[0], neg)


def kernel(query_vector, node_nums, m_A):
    B, Q, D = query_vector.shape
    N = m_A.shape[1]
    nb = pl.cdiv(N, _BLK_N)

    def q_map(b, j, nn_ref):
        return (b, 0, 0)

    def m_map(b, j, nn_ref):
        return (b, j, 0)

    def o_map(b, j, nn_ref):
        return (b, 0, j)

    grid_spec = pltpu.PrefetchScalarGridSpec(
        num_scalar_prefetch=1,
        grid=(B, nb),
        in_specs=[
            pl.BlockSpec((1, Q, D), q_map),
            pl.BlockSpec((1, _BLK_N, D), m_map),
        ],
        out_specs=pl.BlockSpec((1, Q, _BLK_N), o_map),
    )
    return pl.pallas_call(
        _body,
        grid_spec=grid_spec,
        out_shape=jax.ShapeDtypeStruct((B, Q, N), jnp.float32),
        compiler_params=pltpu.CompilerParams(
            dimension_semantics=("parallel", "arbitrary"),
        ),
    )(node_nums.astype(jnp.int32), query_vector, m_A)
